# baseline (speedup 1.0000x reference)
import functools

import jax
import jax.numpy as jnp
import numpy as np
from jax import lax
from jax.experimental import pallas as pl
from jax.experimental.pallas import tpu as pltpu
from jax.experimental.pallas import tpu_sc as plsc

N = 50000
C = 128
FE = 9
G = 32
EPS = 1e-5

NC = 2
NS = 16
HC = C // 2

TB = 50048
NP = 51200
RWS = NP // NS
BLK = 40
NBLK = RWS // BLK
IBLK = FE * BLK

_RBLK = 2000
_RGRID = N // _RBLK


def _stats_body(lv_ref, sum_ref, sq_ref):
    i = pl.program_id(0)

    @pl.when(i == 0)
    def _():
        sum_ref[...] = jnp.zeros_like(sum_ref)
        sq_ref[...] = jnp.zeros_like(sq_ref)

    blk = lv_ref[...]
    sum_ref[...] += jnp.sum(blk, axis=0, keepdims=True)
    sq_ref[...] += jnp.sum(blk * blk, axis=0, keepdims=True)


def _stats(lv):
    return pl.pallas_call(
        _stats_body,
        grid=(_RGRID,),
        in_specs=[pl.BlockSpec((_RBLK, C), lambda i: (i, 0))],
        out_specs=[
            pl.BlockSpec((1, C), lambda i: (0, 0)),
            pl.BlockSpec((1, C), lambda i: (0, 0)),
        ],
        out_shape=[
            jax.ShapeDtypeStruct((1, C), jnp.float32),
            jax.ShapeDtypeStruct((1, C), jnp.float32),
        ],
    )(lv)


_GM = np.kron(np.eye(G, dtype=np.float32),
              np.ones((C // G, C // G), dtype=np.float32)) / float(N * (C // G))


def _norm_body(lv_ref, sum_ref, sq_ref, gamma_ref, beta_ref, gm_ref,
               x0_ref, x1_ref):
    gm = gm_ref[...]
    mean = jnp.dot(sum_ref[...], gm, preferred_element_type=jnp.float32)
    esq = jnp.dot(sq_ref[...], gm, preferred_element_type=jnp.float32)
    var = esq - mean * mean
    inv = lax.rsqrt(var + EPS)
    a = gamma_ref[...] * inv
    bb = beta_ref[...] - mean * a
    xb = jnp.maximum(lv_ref[...] * a + bb, 0.0).astype(jnp.bfloat16)
    u = lax.bitcast_convert_type(xb, jnp.uint16).astype(jnp.int32)
    x0_ref[...] = u[:, 0:32] | (u[:, 32:64] << 16)
    x1_ref[...] = u[:, 64:96] | (u[:, 96:128] << 16)


def _normalize(lv, sums, sq, gamma, beta):
    gm = jnp.asarray(_GM)
    return pl.pallas_call(
        _norm_body,
        grid=(_RGRID,),
        in_specs=[
            pl.BlockSpec((_RBLK, C), lambda i: (i, 0)),
            pl.BlockSpec((1, C), lambda i: (0, 0)),
            pl.BlockSpec((1, C), lambda i: (0, 0)),
            pl.BlockSpec((1, C), lambda i: (0, 0)),
            pl.BlockSpec((1, C), lambda i: (0, 0)),
            pl.BlockSpec((C, C), lambda i: (0, 0)),
        ],
        out_specs=[
            pl.BlockSpec((_RBLK, HC // 2), lambda i: (i, 0)),
            pl.BlockSpec((_RBLK, HC // 2), lambda i: (i, 0)),
        ],
        out_shape=[
            jax.ShapeDtypeStruct((TB, HC // 2), jnp.int32),
            jax.ShapeDtypeStruct((TB, HC // 2), jnp.int32),
        ],
    )(lv, sums, sq, gamma.reshape(1, C), beta.reshape(1, C), gm)


def _sc_body(xt0_hbm, xt1_hbm, idx_hbm, w_hbm, bias_hbm, out_hbm,
             tab_s, idx_a, idx_b, taps_a, taps_b, out_a, out_b, w_v, bias_v,
             gsa, gsb, isa, isb, osem):
    c = lax.axis_index("c")
    s = lax.axis_index("s")
    tpt = TB // NS

    @pl.when(c == 0)
    def _():
        pltpu.sync_copy(xt0_hbm.at[pl.ds(s * tpt, tpt)],
                        tab_s.at[pl.ds(s * tpt, tpt)])

    @pl.when(c == 1)
    def _():
        pltpu.sync_copy(xt1_hbm.at[pl.ds(s * tpt, tpt)],
                        tab_s.at[pl.ds(s * tpt, tpt)])

    pltpu.sync_copy(w_hbm.at[c], w_v)
    pltpu.sync_copy(bias_hbm.at[c], bias_v)
    plsc.subcore_barrier()

    idx_refs = (idx_a, idx_b)
    taps_refs = (taps_a, taps_b)
    out_refs = (out_a, out_b)
    gsems = (gsa, gsb)
    isems = (isa, isb)
    gblk0 = s * NBLK

    def i_copy(bi, slot):
        return pltpu.make_async_copy(
            idx_hbm.at[pl.ds((gblk0 + bi) * IBLK, IBLK)],
            idx_refs[slot], isems[slot])

    def g_copy(slot, k):
        return pltpu.make_async_copy(
            tab_s.at[idx_refs[slot].at[pl.ds(k * BLK, BLK)]],
            taps_refs[slot].at[pl.ds(k * BLK, BLK)],
            gsems[slot])

    def issue_g(slot):
        for k in range(FE):
            g_copy(slot, k).start()

    def wait_g(slot):
        for k in range(FE):
            g_copy(slot, k).wait()

    def out_copy(bi, slot):
        return pltpu.make_async_copy(
            out_refs[slot],
            out_hbm.at[pl.ds(s * RWS + bi * BLK, BLK), pl.ds(c * HC, HC)],
            osem)

    def can_write(bi):
        return s * RWS + bi * BLK < N

    wr = [[w_v[f, pl.ds(j * 32, 32)] for j in range(HC // 32)]
          for f in range(FE)]
    bias_bf = [bias_v[0, pl.ds(j * 32, 32)] for j in range(HC // 32)]

    def compute(slot):
        taps = taps_refs[slot]
        out_r = out_refs[slot]

        def rowfn(r, off):
            for j in range(HC // 32):
                ps = [plsc.bitcast(taps[off + f, pl.ds(j * 16, 16)],
                                   jnp.bfloat16) * wr[f][j]
                      for f in range(FE)] + [bias_bf[j]]
                while len(ps) > 1:
                    nxt = [ps[k] + ps[k + 1] for k in range(0, len(ps) - 1, 2)]
                    if len(ps) % 2:
                        nxt.append(ps[-1])
                    ps = nxt
                lo, hi = plsc.unpack(ps[0], format=plsc.PackFormat.INTERLEAVED)
                out_r[r, pl.ds(j * 16, 16)] = lo
                out_r[r, pl.ds(32 + j * 16, 16)] = hi
            return off + FE

        lax.fori_loop(0, BLK, rowfn, 0, unroll=4)

    i_copy(0, 0).start()
    i_copy(1, 1).start()
    i_copy(0, 0).wait()
    issue_g(0)
    i_copy(1, 1).wait()
    issue_g(1)

    def body2(i2, carry):
        b0 = i2 * 2
        b1 = b0 + 1

        wait_g(0)

        @pl.when(b0 + 2 < NBLK)
        def _():
            i_copy(b0 + 2, 0).start()

        @pl.when(jnp.logical_and(b0 >= 2, can_write(b0 - 2)))
        def _():
            out_copy(b0 - 2, 0).wait()

        compute(0)

        @pl.when(can_write(b0))
        def _():
            out_copy(b0, 0).start()

        @pl.when(b0 + 2 < NBLK)
        def _():
            i_copy(b0 + 2, 0).wait()
            issue_g(0)

        wait_g(1)

        @pl.when(b1 + 2 < NBLK)
        def _():
            i_copy(b1 + 2, 1).start()

        @pl.when(jnp.logical_and(b1 >= 2, can_write(b1 - 2)))
        def _():
            out_copy(b1 - 2, 1).wait()

        compute(1)

        @pl.when(can_write(b1))
        def _():
            out_copy(b1, 1).start()

        @pl.when(b1 + 2 < NBLK)
        def _():
            i_copy(b1 + 2, 1).wait()
            issue_g(1)

        return carry

    lax.fori_loop(0, NBLK // 2, body2, 0)

    @pl.when(can_write(NBLK - 2))
    def _():
        out_copy(NBLK - 2, 0).wait()

    @pl.when(can_write(NBLK - 1))
    def _():
        out_copy(NBLK - 1, 1).wait()


def _sc_conv(xt0, xt1, idx_blocks, w2, bias2):
    mesh = plsc.VectorSubcoreMesh(core_axis_name="c", subcore_axis_name="s")
    f = pl.kernel(
        _sc_body,
        out_type=jax.ShapeDtypeStruct((N, C), jnp.float32),
        mesh=mesh,
        compiler_params=pltpu.CompilerParams(
            needs_layout_passes=False, use_tc_tiling_on_sc=False),
        scratch_types=[
            pltpu.VMEM_SHARED((TB, HC // 2), jnp.int32),
            pltpu.VMEM((IBLK,), jnp.int32),
            pltpu.VMEM((IBLK,), jnp.int32),
            pltpu.VMEM((IBLK, HC // 2), jnp.int32),
            pltpu.VMEM((IBLK, HC // 2), jnp.int32),
            pltpu.VMEM((BLK, HC), jnp.float32),
            pltpu.VMEM((BLK, HC), jnp.float32),
            pltpu.VMEM((FE, HC), jnp.bfloat16),
            pltpu.VMEM((1, HC), jnp.bfloat16),
            pltpu.SemaphoreType.DMA,
            pltpu.SemaphoreType.DMA,
            pltpu.SemaphoreType.DMA,
            pltpu.SemaphoreType.DMA,
            pltpu.SemaphoreType.DMA,
        ],
    )
    return f(xt0, xt1, idx_blocks, w2, bias2)


def kernel(lv, gamma, beta, weight, bias, neighbor_idx):
    sums, sq = _stats(lv)
    xt0, xt1 = _normalize(lv, sums, sq, gamma, beta)

    pos_p = lax.broadcasted_iota(jnp.int32, ((NP - N) * FE,), 0)
    idx_blocks = jnp.concatenate([neighbor_idx.reshape(-1), pos_p & 4095])

    wb = weight.astype(jnp.bfloat16)
    w2 = (wb.reshape(FE, 2, 2, 32).transpose(1, 0, 3, 2)
          .reshape(2, FE, HC))
    bias2 = (bias.astype(jnp.bfloat16).reshape(2, 2, 32)
             .transpose(0, 2, 1).reshape(2, 1, HC))
    return _sc_conv(xt0, xt1, idx_blocks, w2, bias2)


# speedup vs baseline: 1.0059x; 1.0059x over previous
import functools

import jax
import jax.numpy as jnp
import numpy as np
from jax import lax
from jax.experimental import pallas as pl
from jax.experimental.pallas import tpu as pltpu
from jax.experimental.pallas import tpu_sc as plsc

N = 50000
C = 128
FE = 9
G = 32
EPS = 1e-5

NC = 2
NS = 16
HC = C // 2

TB = 50048
NP = 51200
RWS = NP // NS
BLK = 40
NBLK = RWS // BLK
IBLK = FE * BLK

_RBLK = 2000
_RGRID = N // _RBLK


def _stats_body(lv_ref, sum_ref, sq_ref):
    i = pl.program_id(0)

    @pl.when(i == 0)
    def _():
        sum_ref[...] = jnp.zeros_like(sum_ref)
        sq_ref[...] = jnp.zeros_like(sq_ref)

    blk = lv_ref[...]
    sum_ref[...] += jnp.sum(blk, axis=0, keepdims=True)
    sq_ref[...] += jnp.sum(blk * blk, axis=0, keepdims=True)


def _stats(lv):
    return pl.pallas_call(
        _stats_body,
        grid=(_RGRID,),
        in_specs=[pl.BlockSpec((_RBLK, C), lambda i: (i, 0))],
        out_specs=[
            pl.BlockSpec((1, C), lambda i: (0, 0)),
            pl.BlockSpec((1, C), lambda i: (0, 0)),
        ],
        out_shape=[
            jax.ShapeDtypeStruct((1, C), jnp.float32),
            jax.ShapeDtypeStruct((1, C), jnp.float32),
        ],
    )(lv)


_GM = np.kron(np.eye(G, dtype=np.float32),
              np.ones((C // G, C // G), dtype=np.float32)) / float(N * (C // G))


def _norm_body(lv_ref, sum_ref, sq_ref, gamma_ref, beta_ref, gm_ref,
               x0_ref, x1_ref):
    gm = gm_ref[...]
    mean = jnp.dot(sum_ref[...], gm, preferred_element_type=jnp.float32)
    esq = jnp.dot(sq_ref[...], gm, preferred_element_type=jnp.float32)
    var = esq - mean * mean
    inv = lax.rsqrt(var + EPS)
    a = gamma_ref[...] * inv
    bb = beta_ref[...] - mean * a
    xb = jnp.maximum(lv_ref[...] * a + bb, 0.0).astype(jnp.bfloat16)
    u = lax.bitcast_convert_type(xb, jnp.uint16).astype(jnp.int32)
    x0_ref[...] = u[:, 0:32] | (u[:, 32:64] << 16)
    x1_ref[...] = u[:, 64:96] | (u[:, 96:128] << 16)


def _normalize(lv, sums, sq, gamma, beta):
    gm = jnp.asarray(_GM)
    return pl.pallas_call(
        _norm_body,
        grid=(_RGRID,),
        in_specs=[
            pl.BlockSpec((_RBLK, C), lambda i: (i, 0)),
            pl.BlockSpec((1, C), lambda i: (0, 0)),
            pl.BlockSpec((1, C), lambda i: (0, 0)),
            pl.BlockSpec((1, C), lambda i: (0, 0)),
            pl.BlockSpec((1, C), lambda i: (0, 0)),
            pl.BlockSpec((C, C), lambda i: (0, 0)),
        ],
        out_specs=[
            pl.BlockSpec((_RBLK, HC // 2), lambda i: (i, 0)),
            pl.BlockSpec((_RBLK, HC // 2), lambda i: (i, 0)),
        ],
        out_shape=[
            jax.ShapeDtypeStruct((TB, HC // 2), jnp.int32),
            jax.ShapeDtypeStruct((TB, HC // 2), jnp.int32),
        ],
    )(lv, sums, sq, gamma.reshape(1, C), beta.reshape(1, C), gm)


def _sc_body(xt0_hbm, xt1_hbm, idx_hbm, w_hbm, bias_hbm, out_hbm,
             tab_s, idx_a, idx_b, taps_a, taps_b, out_a, out_b, w_v, bias_v,
             gsa, gsb, isa, isb, osem):
    c = lax.axis_index("c")
    s = lax.axis_index("s")
    tpt = TB // NS

    @pl.when(c == 0)
    def _():
        pltpu.sync_copy(xt0_hbm.at[pl.ds(s * tpt, tpt)],
                        tab_s.at[pl.ds(s * tpt, tpt)])

    @pl.when(c == 1)
    def _():
        pltpu.sync_copy(xt1_hbm.at[pl.ds(s * tpt, tpt)],
                        tab_s.at[pl.ds(s * tpt, tpt)])

    pltpu.sync_copy(w_hbm.at[c], w_v)
    pltpu.sync_copy(bias_hbm.at[c], bias_v)
    plsc.subcore_barrier()

    idx_refs = (idx_a, idx_b)
    taps_refs = (taps_a, taps_b)
    out_refs = (out_a, out_b)
    gsems = (gsa, gsb)
    isems = (isa, isb)
    gblk0 = s * NBLK

    def i_copy(bi, slot):
        return pltpu.make_async_copy(
            idx_hbm.at[pl.ds((gblk0 + bi) * IBLK, IBLK)],
            idx_refs[slot], isems[slot])

    def g_copy(slot, k):
        return pltpu.make_async_copy(
            tab_s.at[idx_refs[slot].at[pl.ds(k * BLK, BLK)]],
            taps_refs[slot].at[pl.ds(k * BLK, BLK)],
            gsems[slot])

    def issue_g(slot):
        for k in range(FE):
            g_copy(slot, k).start()

    def wait_g(slot):
        for k in range(FE):
            g_copy(slot, k).wait()

    def out_copy(bi, slot):
        return pltpu.make_async_copy(
            out_refs[slot],
            out_hbm.at[pl.ds(s * RWS + bi * BLK, BLK), pl.ds(c * HC, HC)],
            osem)

    def can_write(bi):
        return s * RWS + bi * BLK < N

    wr = [[w_v[f, pl.ds(j * 32, 32)] for j in range(HC // 32)]
          for f in range(FE)]
    bias_bf = [bias_v[0, pl.ds(j * 32, 32)] for j in range(HC // 32)]

    def compute(slot):
        taps = taps_refs[slot]
        out_r = out_refs[slot]

        def rowfn(r, off):
            for j in range(HC // 32):
                ps = [plsc.bitcast(taps[off + f, pl.ds(j * 16, 16)],
                                   jnp.bfloat16) * wr[f][j]
                      for f in range(FE)] + [bias_bf[j]]
                while len(ps) > 1:
                    nxt = [ps[k] + ps[k + 1] for k in range(0, len(ps) - 1, 2)]
                    if len(ps) % 2:
                        nxt.append(ps[-1])
                    ps = nxt
                lo, hi = plsc.unpack(ps[0], format=plsc.PackFormat.INTERLEAVED)
                out_r[r, pl.ds(j * 16, 16)] = lo
                out_r[r, pl.ds(32 + j * 16, 16)] = hi
            return off + FE

        lax.fori_loop(0, BLK, rowfn, 0, unroll=2)

    i_copy(0, 0).start()
    i_copy(1, 1).start()
    i_copy(0, 0).wait()
    issue_g(0)
    i_copy(1, 1).wait()
    issue_g(1)

    def body2(i2, carry):
        b0 = i2 * 2
        b1 = b0 + 1

        wait_g(0)

        @pl.when(b0 + 2 < NBLK)
        def _():
            i_copy(b0 + 2, 0).start()

        @pl.when(jnp.logical_and(b0 >= 2, can_write(b0 - 2)))
        def _():
            out_copy(b0 - 2, 0).wait()

        compute(0)

        @pl.when(can_write(b0))
        def _():
            out_copy(b0, 0).start()

        @pl.when(b0 + 2 < NBLK)
        def _():
            i_copy(b0 + 2, 0).wait()
            issue_g(0)

        wait_g(1)

        @pl.when(b1 + 2 < NBLK)
        def _():
            i_copy(b1 + 2, 1).start()

        @pl.when(jnp.logical_and(b1 >= 2, can_write(b1 - 2)))
        def _():
            out_copy(b1 - 2, 1).wait()

        compute(1)

        @pl.when(can_write(b1))
        def _():
            out_copy(b1, 1).start()

        @pl.when(b1 + 2 < NBLK)
        def _():
            i_copy(b1 + 2, 1).wait()
            issue_g(1)

        return carry

    lax.fori_loop(0, NBLK // 2, body2, 0)

    @pl.when(can_write(NBLK - 2))
    def _():
        out_copy(NBLK - 2, 0).wait()

    @pl.when(can_write(NBLK - 1))
    def _():
        out_copy(NBLK - 1, 1).wait()


def _sc_conv(xt0, xt1, idx_blocks, w2, bias2):
    mesh = plsc.VectorSubcoreMesh(core_axis_name="c", subcore_axis_name="s")
    f = pl.kernel(
        _sc_body,
        out_type=jax.ShapeDtypeStruct((N, C), jnp.float32),
        mesh=mesh,
        compiler_params=pltpu.CompilerParams(
            needs_layout_passes=False, use_tc_tiling_on_sc=False),
        scratch_types=[
            pltpu.VMEM_SHARED((TB, HC // 2), jnp.int32),
            pltpu.VMEM((IBLK,), jnp.int32),
            pltpu.VMEM((IBLK,), jnp.int32),
            pltpu.VMEM((IBLK, HC // 2), jnp.int32),
            pltpu.VMEM((IBLK, HC // 2), jnp.int32),
            pltpu.VMEM((BLK, HC), jnp.float32),
            pltpu.VMEM((BLK, HC), jnp.float32),
            pltpu.VMEM((FE, HC), jnp.bfloat16),
            pltpu.VMEM((1, HC), jnp.bfloat16),
            pltpu.SemaphoreType.DMA,
            pltpu.SemaphoreType.DMA,
            pltpu.SemaphoreType.DMA,
            pltpu.SemaphoreType.DMA,
            pltpu.SemaphoreType.DMA,
        ],
    )
    return f(xt0, xt1, idx_blocks, w2, bias2)


def kernel(lv, gamma, beta, weight, bias, neighbor_idx):
    sums, sq = _stats(lv)
    xt0, xt1 = _normalize(lv, sums, sq, gamma, beta)

    pos_p = lax.broadcasted_iota(jnp.int32, ((NP - N) * FE,), 0)
    idx_blocks = jnp.concatenate([neighbor_idx.reshape(-1), pos_p & 4095])

    wb = weight.astype(jnp.bfloat16)
    w2 = (wb.reshape(FE, 2, 2, 32).transpose(1, 0, 3, 2)
          .reshape(2, FE, HC))
    bias2 = (bias.astype(jnp.bfloat16).reshape(2, 2, 32)
             .transpose(0, 2, 1).reshape(2, 1, HC))
    return _sc_conv(xt0, xt1, idx_blocks, w2, bias2)
